# baseline (speedup 1.0000x reference)
import functools

import jax
import jax.numpy as jnp
from jax import lax
from jax.experimental import pallas as pl
from jax.experimental.pallas import tpu as pltpu
from jax.experimental.pallas import tpu_sc as plsc

_NUM_WORKERS = 32
_CHUNK = 200
_LANES = 16
_GSPLIT = 104
_IBUF = 208


def _embed_body(max_len, per_worker, idx_hbm, tbl2_hbm, pos_hbm, out_hbm,
                idxb0, idxb1, gidx0, gidx1, offs0, offs1,
                rows0, rows1, outb, pos_v,
                isem0, isem1, gsem0, gsem1, osem):
    d = pos_v.shape[1]
    n_dsl = d // _LANES
    n_chunks = per_worker // _CHUNK
    n_grp = _IBUF // _LANES
    wid = lax.axis_index("s") * 2 + lax.axis_index("c")
    base = wid * per_worker
    scale = jnp.float32(float(d) ** 0.5)

    bufs = (rows0, rows1)
    idxbs = (idxb0, idxb1)
    gidxs = (gidx0, gidx1)
    offss = (offs0, offs1)
    isems = (isem0, isem1)
    gsems = (gsem0, gsem1)

    pltpu.sync_copy(pos_hbm, pos_v)

    def load_idx(c, slot):
        pltpu.async_copy(idx_hbm.at[pl.ds(base + c * _CHUNK, _CHUNK)],
                         idxbs[slot].at[pl.ds(0, _CHUNK)], isems[slot])

    def wait_idx(slot):
        pltpu.make_async_copy(idx_hbm.at[pl.ds(0, _CHUNK)],
                              idxbs[slot].at[pl.ds(0, _CHUNK)],
                              isems[slot]).wait()

    def prime(c, buf, slot):
        for k in range(n_grp):
            sl = pl.ds(k * _LANES, _LANES)
            v = idxbs[slot][sl]
            gidxs[buf][sl] = v >> 1
            offss[buf][k] = (v & 1) << 6
        pltpu.async_copy(tbl2_hbm.at[gidxs[buf].at[pl.ds(0, _GSPLIT)]],
                         bufs[buf].at[pl.ds(0, _GSPLIT)], gsems[buf])
        pltpu.async_copy(tbl2_hbm.at[gidxs[buf].at[pl.ds(_GSPLIT,
                                                         _CHUNK - _GSPLIT)]],
                         bufs[buf].at[pl.ds(_GSPLIT, _CHUNK - _GSPLIT)],
                         gsems[buf])

    def drain_out():
        pltpu.make_async_copy(
            outb, out_hbm.at[pl.ds(0, _CHUNK)], osem).wait()

    def wait_gather(buf):
        pltpu.make_async_copy(
            tbl2_hbm.at[pl.ds(0, _CHUNK)], bufs[buf], gsems[buf]).wait()

    def do_rows(buf, g, offv, lanes):
        for j in range(lanes):
            r = g * _LANES + j
            off = offv[j]
            for s in range(n_dsl):
                sl = pl.ds(s * _LANES, _LANES)
                src = pl.ds(off + s * _LANES, _LANES)
                outb[r, sl] = bufs[buf][r, src] * scale + pos_v[r, sl]

    def compute(buf):
        full = _CHUNK // _LANES
        tail = _CHUNK % _LANES

        @plsc.parallel_loop(0, full, unroll=2)
        def _(g):
            do_rows(buf, g, offss[buf][g], _LANES)

        if tail:
            do_rows(buf, full, offss[buf][full], tail)

    load_idx(0, 0)
    load_idx(1, 1)
    wait_idx(0)
    prime(0, 0, 0)
    load_idx(2, 0)

    def pair_body(i, _):
        for b in range(2):
            c = 2 * i + b
            nslot = 1 - b

            @pl.when(c + 1 < n_chunks)
            def _():
                wait_idx(nslot)
                prime(c + 1, 1 - b, nslot)

            @pl.when(c + 3 < n_chunks)
            def _():
                load_idx(c + 3, nslot)

            wait_gather(b)

            @pl.when(c >= 1)
            def _():
                drain_out()

            compute(b)
            pltpu.async_copy(
                outb, out_hbm.at[pl.ds(base + c * _CHUNK, _CHUNK)], osem)
        return 0

    lax.fori_loop(0, n_chunks // 2, pair_body, 0)
    drain_out()


def kernel(item_id, item_table, pos_table):
    batch, max_len = item_id.shape
    d = item_table.shape[1]
    n_flat = batch * max_len
    per_worker = n_flat // _NUM_WORKERS

    idx_flat = item_id.reshape(n_flat)
    tbl2 = item_table.reshape(item_table.shape[0] // 2, 2 * d)

    mesh = plsc.VectorSubcoreMesh(core_axis_name="c", subcore_axis_name="s")
    body = functools.partial(_embed_body, max_len, per_worker)
    out = pl.kernel(
        body,
        out_type=jax.ShapeDtypeStruct((n_flat, d), jnp.float32),
        scratch_types=[
            pltpu.VMEM((_IBUF,), jnp.int32),
            pltpu.VMEM((_IBUF,), jnp.int32),
            pltpu.VMEM((_IBUF,), jnp.int32),
            pltpu.VMEM((_IBUF,), jnp.int32),
            pltpu.VMEM((_IBUF // _LANES, _LANES), jnp.int32),
            pltpu.VMEM((_IBUF // _LANES, _LANES), jnp.int32),
            pltpu.VMEM((_CHUNK, 2 * d), jnp.float32),
            pltpu.VMEM((_CHUNK, 2 * d), jnp.float32),
            pltpu.VMEM((_CHUNK, d), jnp.float32),
            pltpu.VMEM((max_len, d), jnp.float32),
            pltpu.SemaphoreType.DMA,
            pltpu.SemaphoreType.DMA,
            pltpu.SemaphoreType.DMA,
            pltpu.SemaphoreType.DMA,
            pltpu.SemaphoreType.DMA,
        ],
        mesh=mesh,
    )(idx_flat, tbl2, pos_table)
    return out.reshape(batch, max_len, d)


# speedup vs baseline: 1.2340x; 1.2340x over previous
import functools

import jax
import jax.numpy as jnp
from jax import lax
from jax.experimental import pallas as pl
from jax.experimental.pallas import tpu as pltpu
from jax.experimental.pallas import tpu_sc as plsc

_NUM_WORKERS = 32
_CHUNK = 200
_LANES = 16
_GSPLIT = 104
_IBUF = 208
_UNROLL = 8


def _embed_body(max_len, per_worker, idx_hbm, tblp_hbm, pos_hbm, out_hbm,
                idxb0, idxb1, rows0, rows1, outb, pos_v,
                isem0, isem1, gsem0, gsem1, osem):
    d = pos_v.shape[1]
    n_dsl = d // _LANES
    n_chunks = per_worker // _CHUNK
    wid = lax.axis_index("s") * 2 + lax.axis_index("c")
    base = wid * per_worker
    scale = jnp.float32(float(d) ** 0.5)

    bufs = (rows0, rows1)
    idxbs = (idxb0, idxb1)
    isems = (isem0, isem1)
    gsems = (gsem0, gsem1)

    pltpu.sync_copy(pos_hbm, pos_v)

    def load_idx(c, slot):
        pltpu.async_copy(idx_hbm.at[pl.ds(base + c * _CHUNK, _CHUNK)],
                         idxbs[slot].at[pl.ds(0, _CHUNK)], isems[slot])

    def wait_idx(slot):
        pltpu.make_async_copy(idx_hbm.at[pl.ds(0, _CHUNK)],
                              idxbs[slot].at[pl.ds(0, _CHUNK)],
                              isems[slot]).wait()

    def prime(c, buf, slot):
        pltpu.async_copy(tblp_hbm.at[idxbs[slot].at[pl.ds(0, _GSPLIT)]],
                         bufs[buf].at[pl.ds(0, _GSPLIT)], gsems[buf])
        pltpu.async_copy(tblp_hbm.at[idxbs[slot].at[pl.ds(_GSPLIT,
                                                          _CHUNK - _GSPLIT)]],
                         bufs[buf].at[pl.ds(_GSPLIT, _CHUNK - _GSPLIT)],
                         gsems[buf])

    def drain_out():
        pltpu.make_async_copy(
            outb, out_hbm.at[pl.ds(0, _CHUNK)], osem).wait()

    def wait_gather(buf):
        pltpu.make_async_copy(
            tblp_hbm.at[pl.ds(0, _CHUNK)], bufs[buf], gsems[buf]).wait()

    def compute(buf):
        @plsc.parallel_loop(0, _CHUNK, unroll=_UNROLL)
        def _(r):
            for s in range(n_dsl):
                sl = pl.ds(s * _LANES, _LANES)
                outb[r, sl] = bufs[buf][r, sl] * scale + pos_v[r, sl]

    load_idx(0, 0)
    load_idx(1, 1)
    wait_idx(0)
    prime(0, 0, 0)

    def pair_body(i, _):
        for b in range(2):
            c = 2 * i + b
            nslot = 1 - b

            @pl.when(c + 1 < n_chunks)
            def _():
                wait_idx(nslot)
                prime(c + 1, 1 - b, nslot)

            wait_gather(b)

            @pl.when(c + 2 < n_chunks)
            def _():
                load_idx(c + 2, b)

            @pl.when(c >= 1)
            def _():
                drain_out()

            compute(b)
            pltpu.async_copy(
                outb, out_hbm.at[pl.ds(base + c * _CHUNK, _CHUNK)], osem)
        return 0

    lax.fori_loop(0, n_chunks // 2, pair_body, 0)
    drain_out()


def kernel(item_id, item_table, pos_table):
    batch, max_len = item_id.shape
    d = item_table.shape[1]
    n_flat = batch * max_len
    per_worker = n_flat // _NUM_WORKERS

    idx_flat = item_id.reshape(n_flat)
    tblp = jnp.pad(item_table, ((0, 0), (0, d)))

    mesh = plsc.VectorSubcoreMesh(core_axis_name="c", subcore_axis_name="s")
    body = functools.partial(_embed_body, max_len, per_worker)
    out = pl.kernel(
        body,
        out_type=jax.ShapeDtypeStruct((n_flat, d), jnp.float32),
        scratch_types=[
            pltpu.VMEM((_IBUF,), jnp.int32),
            pltpu.VMEM((_IBUF,), jnp.int32),
            pltpu.VMEM((_CHUNK, 2 * d), jnp.float32),
            pltpu.VMEM((_CHUNK, 2 * d), jnp.float32),
            pltpu.VMEM((_CHUNK, d), jnp.float32),
            pltpu.VMEM((max_len, d), jnp.float32),
            pltpu.SemaphoreType.DMA,
            pltpu.SemaphoreType.DMA,
            pltpu.SemaphoreType.DMA,
            pltpu.SemaphoreType.DMA,
            pltpu.SemaphoreType.DMA,
        ],
        mesh=mesh,
    )(idx_flat, tblp, pos_table)
    return out.reshape(batch, max_len, d)
